# baseline (speedup 1.0000x reference)
import functools

import jax
import jax.numpy as jnp
from jax import lax
from jax.experimental import pallas as pl
from jax.experimental.pallas import tpu as pltpu, tpu_sc as plsc

_BB = 8
_G = 4
_NCHUNK = 16


def _sc_route(position_logits, base, gb):
    B, N, _ = position_logits.shape
    info = plsc.get_sparse_core_info()
    NC, NS, L = info.num_cores, info.num_subcores, info.num_lanes
    NW = NC * NS
    per_w = gb // NW
    mesh = plsc.VectorSubcoreMesh(core_axis_name="c", subcore_axis_name="s")

    @functools.partial(
        pl.kernel,
        out_type=(
            jax.ShapeDtypeStruct((gb, N), jnp.float32),
            jax.ShapeDtypeStruct((gb, N), jnp.float32),
        ),
        mesh=mesh,
        scratch_types=[
            pltpu.VMEM((N, N), jnp.float32),
            pltpu.VMEM((N,), jnp.float32),
            pltpu.VMEM((N,), jnp.float32),
        ],
        name=f"sc_route_g{base}",
    )
    def route_kernel(logits_hbm, conf_hbm, preds_hbm, l_v, c_v, p_v):
        wid = lax.axis_index("s") * NC + lax.axis_index("c")
        for j in range(per_w):
            rb = wid * per_w + j
            pltpu.sync_copy(logits_hbm.at[base + rb], l_v)

            def max_body(i, carry):
                ms, ps = carry
                ivec = jnp.zeros((L,), jnp.int32) + i
                new_ms = []
                new_ps = []
                for c in range(_NCHUNK):
                    x = l_v[i, pl.ds(c * L, L)]
                    gt = x > ms[c]
                    new_ms.append(jnp.where(gt, x, ms[c]))
                    new_ps.append(jnp.where(gt, ivec, ps[c]))
                return tuple(new_ms), tuple(new_ps)

            init = (
                tuple(jnp.full((L,), -jnp.inf, jnp.float32) for _ in range(_NCHUNK)),
                tuple(jnp.zeros((L,), jnp.int32) for _ in range(_NCHUNK)),
            )
            ms, ps = lax.fori_loop(0, N, max_body, init)

            def sum_body(i, ss):
                return tuple(
                    ss[c] + jnp.exp(l_v[i, pl.ds(c * L, L)] - ms[c])
                    for c in range(_NCHUNK)
                )
            zinit = tuple(jnp.zeros((L,), jnp.float32) for _ in range(_NCHUNK))
            ss = lax.fori_loop(0, N, sum_body, zinit)

            for c in range(_NCHUNK):
                c_v[pl.ds(c * L, L)] = 1.0 / ss[c]
                p_v[pl.ds(c * L, L)] = ps[c].astype(jnp.float32)

            pltpu.sync_copy(c_v, conf_hbm.at[rb])
            pltpu.sync_copy(p_v, preds_hbm.at[rb])

    return route_kernel(position_logits)


def _tc_body(*refs):
    if len(refs) == 4:
        _, feat_ref, preds_ref, img_ref = refs
    else:
        feat_ref, preds_ref, img_ref = refs
    BB, N, D = feat_ref.shape
    ii = jax.lax.broadcasted_iota(jnp.int32, (N, N), 0)
    pp = jax.lax.broadcasted_iota(jnp.int32, (N, N), 1)

    for b in range(_BB):
        F = feat_ref[b]
        preds = preds_ref[b].astype(jnp.int32)

        F1 = ii == preds
        lastn = jnp.max(jnp.where(F1, pp, -1), axis=1)
        M = (lastn[:, None] == pp).astype(jnp.float32)

        interior = (M[:-2] + M[1:-1] + M[2:]) * (1.0 / 3.0)
        M2 = jnp.concatenate([M[0:1], interior, M[N - 1:N]], axis=0)

        img_ref[b] = jax.lax.dot_general(
            F.astype(jnp.bfloat16), M2.astype(jnp.bfloat16),
            dimension_numbers=(((0,), (1,)), ((), ())),
            preferred_element_type=jnp.float32,
        )


def _tc_img_group(img_prev, features, preds_g, base, gb):
    B, N, D = features.shape
    goff = base // _BB
    data_specs = [
        pl.BlockSpec((_BB, N, D), lambda b: (goff + b, 0, 0)),
        pl.BlockSpec((_BB, 1, N), lambda b: (b, 0, 0)),
    ]
    if img_prev is None:
        in_specs, args, aliases = data_specs, (features, preds_g), {}
    else:
        in_specs = [pl.BlockSpec(memory_space=pl.ANY)] + data_specs
        args, aliases = (img_prev, features, preds_g), {0: 0}
    return pl.pallas_call(
        _tc_body,
        grid=(gb // _BB,),
        in_specs=in_specs,
        out_specs=pl.BlockSpec((_BB, D, N), lambda b: (goff + b, 0, 0)),
        out_shape=jax.ShapeDtypeStruct((B, D, N), jnp.float32),
        input_output_aliases=aliases,
        name=f"tc_img_g{base}",
    )(*args)


@jax.jit
def kernel(features, position_logits):
    B, N, D = features.shape
    gb = B // _G

    routed = [_sc_route(position_logits, g * gb, gb) for g in range(_G)]

    img = None
    for g in range(_G):
        conf_g, preds_g = routed[g]
        img = _tc_img_group(img, features, preds_g.reshape(gb, 1, N),
                            g * gb, gb)

    conf = jnp.concatenate([r[0] for r in routed], axis=0)
    g = int(round(N ** 0.5))
    return img.reshape(B, D, g, g), conf


# speedup vs baseline: 1.2189x; 1.2189x over previous
import functools

import jax
import jax.numpy as jnp
from jax import lax
from jax.experimental import pallas as pl
from jax.experimental.pallas import tpu as pltpu, tpu_sc as plsc

_BB = 8
_NCHUNK = 16


def _tc_body(feat_ref, logits_ref, img_ref):
    N = logits_ref.shape[1]
    ii = jax.lax.broadcasted_iota(jnp.int32, (N, N), 0)
    pp = jax.lax.broadcasted_iota(jnp.int32, (N, N), 1)

    for b in range(_BB):
        L = logits_ref[b]
        F = feat_ref[b]

        m = jnp.max(L, axis=0)
        preds = jnp.min(jnp.where(L == m[None, :], ii, N), axis=0)

        F1 = ii == preds[None, :]
        lastn = jnp.max(jnp.where(F1, pp, -1), axis=1)
        M = (lastn[:, None] == pp).astype(jnp.float32)

        interior = (M[:-2] + M[1:-1] + M[2:]) * (1.0 / 3.0)
        M2 = jnp.concatenate([M[0:1], interior, M[N - 1:N]], axis=0)

        img_ref[b] = jax.lax.dot_general(
            F.astype(jnp.bfloat16), M2.astype(jnp.bfloat16),
            dimension_numbers=(((0,), (1,)), ((), ())),
            preferred_element_type=jnp.float32,
        )


def _tc_img(features, position_logits):
    B, N, D = features.shape
    return pl.pallas_call(
        _tc_body,
        grid=(B // _BB,),
        in_specs=[
            pl.BlockSpec((_BB, N, D), lambda b: (b, 0, 0)),
            pl.BlockSpec((_BB, N, N), lambda b: (b, 0, 0)),
        ],
        out_specs=pl.BlockSpec((_BB, D, N), lambda b: (b, 0, 0)),
        out_shape=jax.ShapeDtypeStruct((B, D, N), jnp.float32),
    )(features, position_logits)


def _sc_conf(position_logits):
    B, N, _ = position_logits.shape
    info = plsc.get_sparse_core_info()
    NC, NS, L = info.num_cores, info.num_subcores, info.num_lanes
    NW = NC * NS
    per_w = B // NW
    mesh = plsc.VectorSubcoreMesh(core_axis_name="c", subcore_axis_name="s")

    @functools.partial(
        pl.kernel,
        out_type=jax.ShapeDtypeStruct((B, N), jnp.float32),
        mesh=mesh,
        scratch_types=[
            pltpu.VMEM((N, N), jnp.float32),
            pltpu.VMEM((N,), jnp.float32),
        ],
    )
    def conf_kernel(logits_hbm, conf_hbm, l_v, c_v):
        wid = lax.axis_index("s") * NC + lax.axis_index("c")
        for j in range(per_w):
            b = wid * per_w + j
            pltpu.sync_copy(logits_hbm.at[b], l_v)

            def max_body(i, ms):
                return tuple(
                    jnp.maximum(ms[c], l_v[i, pl.ds(c * L, L)])
                    for c in range(_NCHUNK)
                )
            init = tuple(jnp.full((L,), -jnp.inf, jnp.float32)
                         for _ in range(_NCHUNK))
            ms = lax.fori_loop(0, N, max_body, init)

            def sum_body(i, ss):
                return tuple(
                    ss[c] + jnp.exp(l_v[i, pl.ds(c * L, L)] - ms[c])
                    for c in range(_NCHUNK)
                )
            zinit = tuple(jnp.zeros((L,), jnp.float32) for _ in range(_NCHUNK))
            ss = lax.fori_loop(0, N, sum_body, zinit)

            for c in range(_NCHUNK):
                c_v[pl.ds(c * L, L)] = 1.0 / ss[c]
            pltpu.sync_copy(c_v, conf_hbm.at[b])

    return conf_kernel(position_logits)


@jax.jit
def kernel(features, position_logits):
    B, N, D = features.shape
    img = _tc_img(features, position_logits)
    conf = _sc_conf(position_logits)
    g = int(round(N ** 0.5))
    return img.reshape(B, D, g, g), conf


# speedup vs baseline: 1.2509x; 1.0262x over previous
import functools

import jax
import jax.numpy as jnp
from jax import lax
from jax.experimental import pallas as pl
from jax.experimental.pallas import tpu as pltpu, tpu_sc as plsc

_BB = 8
_NCHUNK = 16


def _tc_body(feat_ref, logits_ref, img_ref, conf_ref):
    N = logits_ref.shape[1]
    ii = jax.lax.broadcasted_iota(jnp.int32, (N, N), 0)
    pp = jax.lax.broadcasted_iota(jnp.int32, (N, N), 1)

    for b in range(_BB):
        L = logits_ref[b]
        F = feat_ref[b]

        m = jnp.max(L, axis=0)
        s = jnp.sum(jnp.exp(L - m[None, :]), axis=0)
        conf_ref[b, 0, :] = 1.0 / s
        preds = jnp.min(jnp.where(L == m[None, :], ii, N), axis=0)

        F1 = ii == preds[None, :]
        lastn = jnp.max(jnp.where(F1, pp, -1), axis=1)
        M = (lastn[:, None] == pp).astype(jnp.float32)

        interior = (M[:-2] + M[1:-1] + M[2:]) * (1.0 / 3.0)
        M2 = jnp.concatenate([M[0:1], interior, M[N - 1:N]], axis=0)

        img_ref[b] = jax.lax.dot_general(
            F.astype(jnp.bfloat16), M2.astype(jnp.bfloat16),
            dimension_numbers=(((0,), (1,)), ((), ())),
            preferred_element_type=jnp.float32,
        )


def _tc_img(features, position_logits):
    B, N, D = features.shape
    return pl.pallas_call(
        _tc_body,
        grid=(B // _BB,),
        in_specs=[
            pl.BlockSpec((_BB, N, D), lambda b: (b, 0, 0)),
            pl.BlockSpec((_BB, N, N), lambda b: (b, 0, 0)),
        ],
        out_specs=[
            pl.BlockSpec((_BB, D, N), lambda b: (b, 0, 0)),
            pl.BlockSpec((_BB, 1, N), lambda b: (b, 0, 0)),
        ],
        out_shape=[
            jax.ShapeDtypeStruct((B, D, N), jnp.float32),
            jax.ShapeDtypeStruct((B, 1, N), jnp.float32),
        ],
    )(features, position_logits)


def _sc_conf(position_logits, nb):
    B, N, _ = position_logits.shape
    info = plsc.get_sparse_core_info()
    NC, NS, L = info.num_cores, info.num_subcores, info.num_lanes
    NW = NC * NS
    per_w = nb // NW
    mesh = plsc.VectorSubcoreMesh(core_axis_name="c", subcore_axis_name="s")

    @functools.partial(
        pl.kernel,
        out_type=jax.ShapeDtypeStruct((nb, N), jnp.float32),
        mesh=mesh,
        scratch_types=[
            pltpu.VMEM((N, N), jnp.float32),
            pltpu.VMEM((N,), jnp.float32),
        ],
    )
    def conf_kernel(logits_hbm, conf_hbm, l_v, c_v):
        wid = lax.axis_index("s") * NC + lax.axis_index("c")
        for j in range(per_w):
            b = wid * per_w + j
            pltpu.sync_copy(logits_hbm.at[b], l_v)

            def max_body(i, ms):
                return tuple(
                    jnp.maximum(ms[c], l_v[i, pl.ds(c * L, L)])
                    for c in range(_NCHUNK)
                )
            init = tuple(jnp.full((L,), -jnp.inf, jnp.float32)
                         for _ in range(_NCHUNK))
            ms = lax.fori_loop(0, N, max_body, init)

            def sum_body(i, ss):
                return tuple(
                    ss[c] + jnp.exp(l_v[i, pl.ds(c * L, L)] - ms[c])
                    for c in range(_NCHUNK)
                )
            zinit = tuple(jnp.zeros((L,), jnp.float32) for _ in range(_NCHUNK))
            ss = lax.fori_loop(0, N, sum_body, zinit)

            for c in range(_NCHUNK):
                c_v[pl.ds(c * L, L)] = 1.0 / ss[c]
            pltpu.sync_copy(c_v, conf_hbm.at[b])

    return conf_kernel(position_logits)


@jax.jit
def kernel(features, position_logits):
    B, N, D = features.shape
    nb = B // 2
    img, conf_tc = _tc_img(features, position_logits)
    conf_sc = _sc_conf(position_logits, nb)
    conf = jnp.concatenate([conf_sc, conf_tc.reshape(B, N)[nb:]], axis=0)
    g = int(round(N ** 0.5))
    return img.reshape(B, D, g, g), conf


# speedup vs baseline: 1.2585x; 1.0061x over previous
import functools

import jax
import jax.numpy as jnp
from jax import lax
from jax.experimental import pallas as pl
from jax.experimental.pallas import tpu as pltpu, tpu_sc as plsc

_BB = 8
_NCHUNK = 16


def _tc_body(feat_ref, logits_ref, img_ref, conf_ref):
    N = logits_ref.shape[1]
    ii = jax.lax.broadcasted_iota(jnp.int32, (N, N), 0)
    pp = jax.lax.broadcasted_iota(jnp.int32, (N, N), 1)

    for b in range(_BB):
        L = logits_ref[b]
        F = feat_ref[b]

        m = jnp.max(L, axis=0)
        s = jnp.sum(jnp.exp(L - m[None, :]), axis=0)
        conf_ref[b, 0, :] = 1.0 / s
        preds = jnp.min(jnp.where(L == m[None, :], ii, N), axis=0)

        F1 = ii == preds[None, :]
        lastn = jnp.max(jnp.where(F1, pp, -1), axis=1)
        M = (lastn[:, None] == pp).astype(jnp.float32)

        interior = (M[:-2] + M[1:-1] + M[2:]) * (1.0 / 3.0)
        M2 = jnp.concatenate([M[0:1], interior, M[N - 1:N]], axis=0)

        img_ref[b] = jax.lax.dot_general(
            F.astype(jnp.bfloat16), M2.astype(jnp.bfloat16),
            dimension_numbers=(((0,), (1,)), ((), ())),
            preferred_element_type=jnp.float32,
        )


def _tc_img(features, position_logits):
    B, N, D = features.shape
    return pl.pallas_call(
        _tc_body,
        grid=(B // _BB,),
        in_specs=[
            pl.BlockSpec((_BB, N, D), lambda b: (b, 0, 0)),
            pl.BlockSpec((_BB, N, N), lambda b: (b, 0, 0)),
        ],
        out_specs=[
            pl.BlockSpec((_BB, D, N), lambda b: (b, 0, 0)),
            pl.BlockSpec((_BB, 1, N), lambda b: (b, 0, 0)),
        ],
        out_shape=[
            jax.ShapeDtypeStruct((B, D, N), jnp.float32),
            jax.ShapeDtypeStruct((B, 1, N), jnp.float32),
        ],
    )(features, position_logits)


def _sc_conf(position_logits, nb):
    B, N, _ = position_logits.shape
    info = plsc.get_sparse_core_info()
    NC, NS, L = info.num_cores, info.num_subcores, info.num_lanes
    NW = NC * NS
    per_w = nb // NW
    mesh = plsc.VectorSubcoreMesh(core_axis_name="c", subcore_axis_name="s")

    @functools.partial(
        pl.kernel,
        out_type=jax.ShapeDtypeStruct((nb, N), jnp.float32),
        mesh=mesh,
        scratch_types=[
            pltpu.VMEM((N, N), jnp.float32),
            pltpu.VMEM((N,), jnp.float32),
        ],
    )
    def conf_kernel(logits_hbm, conf_hbm, l_v, c_v):
        wid = lax.axis_index("s") * NC + lax.axis_index("c")
        for j in range(per_w):
            b = wid * per_w + j
            pltpu.sync_copy(logits_hbm.at[b], l_v)

            def max_body(i, ms):
                return tuple(
                    jnp.maximum(ms[c], l_v[i, pl.ds(c * L, L)])
                    for c in range(_NCHUNK)
                )
            init = tuple(jnp.full((L,), -jnp.inf, jnp.float32)
                         for _ in range(_NCHUNK))
            ms = lax.fori_loop(0, N, max_body, init)

            def sum_body(i, ss):
                return tuple(
                    ss[c] + jnp.exp(l_v[i, pl.ds(c * L, L)] - ms[c])
                    for c in range(_NCHUNK)
                )
            zinit = tuple(jnp.zeros((L,), jnp.float32) for _ in range(_NCHUNK))
            ss = lax.fori_loop(0, N, sum_body, zinit)

            for c in range(_NCHUNK):
                c_v[pl.ds(c * L, L)] = 1.0 / ss[c]
            pltpu.sync_copy(c_v, conf_hbm.at[b])

    return conf_kernel(position_logits)


@jax.jit
def kernel(features, position_logits):
    B, N, D = features.shape
    nb = B // 4
    img, conf_tc = _tc_img(features, position_logits)
    conf_sc = _sc_conf(position_logits, nb)
    conf = jnp.concatenate([conf_sc, conf_tc.reshape(B, N)[nb:]], axis=0)
    g = int(round(N ** 0.5))
    return img.reshape(B, D, g, g), conf
